# baseline (speedup 1.0000x reference)
import functools

import jax
import jax.numpy as jnp
from jax import lax
from jax.experimental import pallas as pl
from jax.experimental.pallas import tpu as pltpu
from jax.experimental.pallas import tpu_sc as plsc

_N = 10000
_E = 320000
_D = 128
_DO = 64

_NC = 2
_NS = 16
_CHUNK = 80
_SUP = 8
_NBUF = 4
_C0 = 192
_C1 = 64
_NSUP0 = _C0 // _SUP
_NSUP1 = _C1 // _SUP
_TOTC = _NS * (_C0 + _C1)
_EPAD = _TOTC * _CHUNK
_NPAD = 10240
_RPT = _NPAD // _NS

_BM = 1024
_BMF = 1000


def _build_spmm():
    mesh = plsc.VectorSubcoreMesh(core_axis_name="c", subcore_axis_name="s")

    @functools.partial(
        pl.kernel,
        out_type=jax.ShapeDtypeStruct((_NC, _NPAD, _D), jnp.float32),
        mesh=mesh,
        scratch_types=[
            pltpu.VMEM((2, _SUP, _CHUNK), jnp.int32),
            pltpu.VMEM((2, _SUP, _CHUNK), jnp.int32),
            pltpu.VMEM((2, _SUP, _CHUNK), jnp.float32),
            [pltpu.VMEM((_CHUNK, _D), jnp.float32)] * _NBUF,
            pltpu.VMEM_SHARED((_NPAD, _D), jnp.float32),
            [pltpu.SemaphoreType.DMA] * _NBUF,
            [pltpu.SemaphoreType.DMA] * _NBUF,
            [pltpu.SemaphoreType.DMA] * 3,
        ],
    )
    def spmm(h_hbm, src_hbm, dst_hbm, w_hbm, z_hbm, out_hbm,
             sidx, didx, wblk, rows, acc_sh, gsem, ssem, isem):
        cid = lax.axis_index("c")
        sid = lax.axis_index("s")
        r0 = sid * _RPT
        base = jnp.where(cid == 0, sid * _C0, _NS * _C0 + sid * _C1)
        nsup = jnp.where(cid == 0, _NSUP0, _NSUP1)

        pltpu.sync_copy(z_hbm.at[pl.ds(r0, _RPT)], acc_sh.at[pl.ds(r0, _RPT)])
        pltpu.sync_copy(src_hbm.at[pl.ds(base, _SUP)], sidx.at[0])
        pltpu.sync_copy(dst_hbm.at[pl.ds(base, _SUP)], didx.at[0])
        pltpu.sync_copy(w_hbm.at[pl.ds(base, _SUP)], wblk.at[0])
        plsc.subcore_barrier()

        def issue_gather(p, r, b):
            pltpu.async_copy(h_hbm.at[sidx.at[p, r]], rows[b], gsem[b])

        def wait_gather(p, r, b):
            pltpu.make_async_copy(h_hbm.at[sidx.at[p, r]], rows[b],
                                  gsem[b]).wait()

        def issue_scatter(p, r, b):
            pltpu.async_copy(rows[b], acc_sh.at[didx.at[p, r]], ssem[b],
                             add=True)

        def wait_scatter(p, r, b):
            pltpu.make_async_copy(rows[b], acc_sh.at[didx.at[p, r]],
                                  ssem[b]).wait()

        def scale(p, k, b):
            rv = rows[b]

            def group_body(g, carry2):
                wv = wblk[p, k, pl.ds(g * 16, 16)]
                bs = g * 16
                for t in range(16):
                    w = wv[t]
                    for j in range(_D // 16):
                        sl = pl.ds(j * 16, 16)
                        rv[bs + t, sl] = rv[bs + t, sl] * w
                return carry2

            lax.fori_loop(0, _CHUNK // 16, group_body, 0)

        issue_gather(0, 0, 0)
        issue_gather(0, 1, 1)

        def super_body(ci, carry):
            p = lax.rem(ci, 2)
            pn = 1 - p
            not_last = ci < nsup - 1

            for k in range(_SUP):
                b = k % _NBUF
                j = (k + 2) % _NBUF

                if k == 0:
                    @pl.when(not_last)
                    def _():
                        nb = pl.ds(base + (ci + 1) * _SUP, _SUP)
                        pltpu.async_copy(src_hbm.at[nb], sidx.at[pn],
                                         isem[0])
                        pltpu.async_copy(dst_hbm.at[nb], didx.at[pn],
                                         isem[1])
                        pltpu.async_copy(w_hbm.at[nb], wblk.at[pn],
                                         isem[2])

                if k < 2:
                    @pl.when(ci >= 1)
                    def _():
                        wait_scatter(pn, k + 6, j)
                    issue_gather(p, k + 2, j)
                elif k < 6:
                    wait_scatter(p, k - 2, j)
                    issue_gather(p, k + 2, j)
                else:
                    if k == 6:
                        @pl.when(not_last)
                        def _():
                            pltpu.make_async_copy(
                                src_hbm.at[pl.ds(base, _SUP)],
                                sidx.at[pn], isem[0]).wait()
                            pltpu.make_async_copy(
                                dst_hbm.at[pl.ds(base, _SUP)],
                                didx.at[pn], isem[1]).wait()
                            pltpu.make_async_copy(
                                w_hbm.at[pl.ds(base, _SUP)],
                                wblk.at[pn], isem[2]).wait()

                    wait_scatter(p, k - 2, j)

                    @pl.when(not_last)
                    def _():
                        issue_gather(pn, k - 6, j)

                wait_gather(p, k, b)
                scale(p, k, b)
                issue_scatter(p, k, b)
            return carry

        lax.fori_loop(0, nsup, super_body, 0)
        p_last = lax.rem(nsup - 1, 2)
        wait_scatter(p_last, _SUP - 2, (_SUP - 2) % _NBUF)
        wait_scatter(p_last, _SUP - 1, (_SUP - 1) % _NBUF)
        plsc.subcore_barrier()
        pltpu.sync_copy(acc_sh.at[pl.ds(r0, _RPT)],
                        out_hbm.at[cid, pl.ds(r0, _RPT)])

    return spmm


_spmm = _build_spmm()


def _mm_body(x_ref, w_ref, o_ref):
    o_ref[...] = jnp.dot(x_ref[...], w_ref[...],
                         preferred_element_type=jnp.float32)


def _mm(x, w):
    return pl.pallas_call(
        _mm_body,
        grid=(_NPAD // _BM,),
        in_specs=[
            pl.BlockSpec((_BM, _D), lambda i: (i, 0)),
            pl.BlockSpec((_D, _D), lambda i: (0, 0)),
        ],
        out_specs=pl.BlockSpec((_BM, _D), lambda i: (i, 0)),
        out_shape=jax.ShapeDtypeStruct((_NPAD, _D), jnp.float32),
    )(x, w)


def _fuse_body(p_ref, w_ref, o_ref):
    h = jnp.maximum(p_ref[0] + p_ref[1], 0.0)
    o_ref[...] = jnp.dot(h, w_ref[...], preferred_element_type=jnp.float32)


def _fuse_relu_mm(p, w):
    return pl.pallas_call(
        _fuse_body,
        grid=(_NPAD // _BM,),
        in_specs=[
            pl.BlockSpec((_NC, _BM, _D), lambda i: (0, i, 0)),
            pl.BlockSpec((_D, _D), lambda i: (0, 0)),
        ],
        out_specs=pl.BlockSpec((_BM, _D), lambda i: (i, 0)),
        out_shape=jax.ShapeDtypeStruct((_NPAD, _D), jnp.float32),
    )(p, w)


def _fin_body(q_ref, mu_ref, lv_ref):
    s = q_ref[0] + q_ref[1]
    m = s[:, :_DO]
    norm = jnp.sqrt(jnp.sum(m * m, axis=1, keepdims=True))
    mu_ref[...] = m / jnp.maximum(norm, 1e-12)
    lv_ref[...] = s[:, _DO:]


def _finalize(q):
    return pl.pallas_call(
        _fin_body,
        grid=(_N // _BMF,),
        in_specs=[pl.BlockSpec((_NC, _BMF, _D), lambda i: (0, i, 0))],
        out_specs=[
            pl.BlockSpec((_BMF, _DO), lambda i: (i, 0)),
            pl.BlockSpec((_BMF, _DO), lambda i: (i, 0)),
        ],
        out_shape=[
            jax.ShapeDtypeStruct((_N, _DO), jnp.float32),
            jax.ShapeDtypeStruct((_N, _DO), jnp.float32),
        ],
    )(q)


def kernel(x, adj, edge_weight, W1, W2, W3):
    pad = _EPAD - _E
    pad_dst = _N + (jnp.arange(pad, dtype=jnp.int32) % (_NPAD - _N))
    src = jnp.concatenate([adj[0], jnp.zeros((pad,), jnp.int32)])
    dst = jnp.concatenate([adj[1], pad_dst])
    ew = jnp.concatenate([edge_weight, jnp.zeros((pad,), jnp.float32)])
    src = src.reshape(_TOTC, _CHUNK)
    dst = dst.reshape(_TOTC, _CHUNK)
    ew = ew.reshape(_TOTC, _CHUNK)
    zeros = jnp.zeros((_NPAD, _D), jnp.float32)
    wcat = jnp.concatenate([W2, W3], axis=1)
    xpad = jnp.concatenate(
        [x, jnp.zeros((_NPAD - _N, _D), jnp.float32)], axis=0)

    xw = _mm(xpad, W1)
    p = _spmm(xw, src, dst, ew, zeros)
    hw = _fuse_relu_mm(p, wcat)
    q = _spmm(hw, src, dst, ew, zeros)
    mu, logvar = _finalize(q)
    return (mu, mu, logvar)
